# Initial kernel scaffold; baseline (speedup 1.0000x reference)
import jax
import jax.numpy as jnp
from jax.experimental import pallas as pl


def kernel(xyz, points, W0, W1):
    raise NotImplementedError("write your pallas kernel here")


# speedup vs baseline: 1.2124x; 1.2124x over previous
import functools

import jax
import jax.numpy as jnp
from jax.experimental import pallas as pl
from jax.experimental.pallas import tpu as pltpu

NPOINT = 2048
NSAMPLE = 32


def _mlp_pool_body(f_ref, c_ref, w1t_ref, o_ref):
    f = f_ref[0]
    c = c_ref[0]
    h1 = f - c[:, None, :]
    h1 = jnp.where(h1 >= 0, h1, 0.1 * h1)
    sb, k, d = h1.shape
    h2 = jnp.dot(h1.reshape(sb * k, d), w1t_ref[...],
                 preferred_element_type=jnp.float32)
    h2 = jnp.where(h2 >= 0, h2, 0.1 * h2)
    o_ref[0] = jnp.max(h2.reshape(sb, k, -1), axis=1)


def _mlp_pool(f, c, w1t, s_block=256):
    b, s, k, d = f.shape
    grid = (b, s // s_block)
    return pl.pallas_call(
        _mlp_pool_body,
        grid=grid,
        in_specs=[
            pl.BlockSpec((1, s_block, k, d), lambda i, j: (i, j, 0, 0)),
            pl.BlockSpec((1, s_block, d), lambda i, j: (i, j, 0)),
            pl.BlockSpec((d, d), lambda i, j: (0, 0)),
        ],
        out_specs=pl.BlockSpec((1, s_block, d), lambda i, j: (i, j, 0)),
        out_shape=jax.ShapeDtypeStruct((b, s, d), jnp.float32),
    )(f, c, w1t)


def kernel(xyz, points, W0, W1):
    B, C, N = xyz.shape
    S, K = NPOINT, NSAMPLE
    xyz_t = jnp.swapaxes(xyz, 1, 2)
    new_xyz = xyz[..., :S]
    new_xyz_t = xyz_t[:, :S]

    W0x = W0[:, :3]
    W0p = W0[:, 3:]
    X1 = jnp.einsum('oc,bcn->bno', W0x, xyz)
    P = X1 + jnp.einsum('od,bdn->bno', W0p, points)
    Cc = X1[:, :S]

    sq = jnp.sum(xyz_t * xyz_t, axis=-1)
    d = (-2.0 * jnp.einsum('bsc,bnc->bsn', new_xyz_t, xyz_t)
         + sq[:, :S, None] + sq[:, None, :])
    _, knn_idx = jax.lax.top_k(-d, K)

    F = jnp.take_along_axis(P[:, None], knn_idx[..., None], axis=2)

    out = _mlp_pool(F, Cc, W1.T)
    return (new_xyz, jnp.swapaxes(out, 1, 2))


# speedup vs baseline: 1.5056x; 1.2419x over previous
import functools

import jax
import jax.numpy as jnp
from jax import lax
from jax.experimental import pallas as pl
from jax.experimental.pallas import tpu as pltpu
from jax.experimental.pallas import tpu_sc as plsc

NPOINT = 2048
NSAMPLE = 32


def _sc_gather_rows(table, idx_flat, chunk=1024):
    M, D = table.shape
    T = idx_flat.shape[0]
    info = plsc.get_sparse_core_info()
    nw = info.num_cores * info.num_subcores
    per_w = T // nw
    assert T % nw == 0 and per_w % chunk == 0

    mesh = plsc.VectorSubcoreMesh(core_axis_name="c", subcore_axis_name="s")

    @functools.partial(
        pl.kernel, mesh=mesh,
        out_type=jax.ShapeDtypeStruct((T, D), jnp.float32),
        compiler_params=pltpu.CompilerParams(use_tc_tiling_on_sc=False),
        scratch_types=[
            pltpu.VMEM((chunk,), jnp.int32),
            pltpu.VMEM((chunk, D), jnp.float32),
            pltpu.SemaphoreType.DMA,
        ],
    )
    def k(table_hbm, idx_hbm, out_hbm, idx_v, rows_v, sem):
        wid = lax.axis_index("s") * info.num_cores + lax.axis_index("c")
        base = wid * per_w

        def body(i, carry):
            off = base + i * chunk
            pltpu.sync_copy(idx_hbm.at[pl.ds(off, chunk)], idx_v)
            pltpu.async_copy(table_hbm.at[idx_v], rows_v, sem).wait()
            pltpu.sync_copy(rows_v, out_hbm.at[pl.ds(off, chunk)])
            return carry

        lax.fori_loop(0, per_w // chunk, body, 0)

    return k(table, idx_flat)


def _mlp_pool_body(f_ref, c_ref, w1t_ref, o_ref):
    f = f_ref[0]
    c = c_ref[0]
    h1 = f - c[:, None, :]
    h1 = jnp.where(h1 >= 0, h1, 0.1 * h1)
    sb, k, d = h1.shape
    h2 = jnp.dot(h1.reshape(sb * k, d), w1t_ref[...],
                 preferred_element_type=jnp.float32)
    h2 = jnp.where(h2 >= 0, h2, 0.1 * h2)
    o_ref[0] = jnp.max(h2.reshape(sb, k, -1), axis=1)


def _mlp_pool(f, c, w1t, s_block=256):
    b, s, k, d = f.shape
    grid = (b, s // s_block)
    return pl.pallas_call(
        _mlp_pool_body,
        grid=grid,
        in_specs=[
            pl.BlockSpec((1, s_block, k, d), lambda i, j: (i, j, 0, 0)),
            pl.BlockSpec((1, s_block, d), lambda i, j: (i, j, 0)),
            pl.BlockSpec((d, d), lambda i, j: (0, 0)),
        ],
        out_specs=pl.BlockSpec((1, s_block, d), lambda i, j: (i, j, 0)),
        out_shape=jax.ShapeDtypeStruct((b, s, d), jnp.float32),
    )(f, c, w1t)


def kernel(xyz, points, W0, W1):
    B, C, N = xyz.shape
    S, K = NPOINT, NSAMPLE
    xyz_t = jnp.swapaxes(xyz, 1, 2)
    new_xyz = xyz[..., :S]
    new_xyz_t = xyz_t[:, :S]

    W0x = W0[:, :3]
    W0p = W0[:, 3:]
    X1 = jnp.einsum('oc,bcn->bno', W0x, xyz)
    P = X1 + jnp.einsum('od,bdn->bno', W0p, points)
    Cc = X1[:, :S]

    sq = jnp.sum(xyz_t * xyz_t, axis=-1)
    d = (-2.0 * jnp.einsum('bsc,bnc->bsn', new_xyz_t, xyz_t)
         + sq[:, :S, None] + sq[:, None, :])
    _, knn_idx = jax.lax.top_k(-d, K)

    gidx = (knn_idx + (jnp.arange(B, dtype=knn_idx.dtype) * N)[:, None, None])
    F = _sc_gather_rows(P.reshape(B * N, -1), gidx.reshape(-1))
    F = F.reshape(B, S, K, -1)

    out = _mlp_pool(F, Cc, W1.T)
    return (new_xyz, jnp.swapaxes(out, 1, 2))


# speedup vs baseline: 16.1677x; 10.7385x over previous
import functools

import jax
import jax.numpy as jnp
from jax import lax
from jax.experimental import pallas as pl
from jax.experimental.pallas import tpu as pltpu
from jax.experimental.pallas import tpu_sc as plsc

NPOINT = 2048
NSAMPLE = 32
SEG = 32
SB = 256
NCHUNK = 2048


def _sc_gather_rows(table, idx_flat, chunk=1024):
    M, D = table.shape
    T = idx_flat.shape[0]
    info = plsc.get_sparse_core_info()
    nw = info.num_cores * info.num_subcores
    per_w = T // nw
    assert T % nw == 0 and per_w % chunk == 0

    mesh = plsc.VectorSubcoreMesh(core_axis_name="c", subcore_axis_name="s")

    @functools.partial(
        pl.kernel, mesh=mesh,
        out_type=jax.ShapeDtypeStruct((T, D), jnp.float32),
        compiler_params=pltpu.CompilerParams(use_tc_tiling_on_sc=False),
        scratch_types=[
            pltpu.VMEM((chunk,), jnp.int32),
            pltpu.VMEM((chunk, D), jnp.float32),
            pltpu.SemaphoreType.DMA,
        ],
    )
    def k(table_hbm, idx_hbm, out_hbm, idx_v, rows_v, sem):
        wid = lax.axis_index("s") * info.num_cores + lax.axis_index("c")
        base = wid * per_w

        def body(i, carry):
            off = base + i * chunk
            pltpu.sync_copy(idx_hbm.at[pl.ds(off, chunk)], idx_v)
            pltpu.async_copy(table_hbm.at[idx_v], rows_v, sem).wait()
            pltpu.sync_copy(rows_v, out_hbm.at[pl.ds(off, chunk)])
            return carry

        lax.fori_loop(0, per_w // chunk, body, 0)

    return k(table, idx_flat)


def _conv1_body(xyz_ref, pts_ref, w0x_ref, w0p_ref, p_ref, c_ref):
    xyz = xyz_ref[0]
    x1 = (w0x_ref[:, 0:1] * xyz[0:1]
          + w0x_ref[:, 1:2] * xyz[1:2]
          + w0x_ref[:, 2:3] * xyz[2:3])
    p = x1 + jnp.dot(w0p_ref[...], pts_ref[0],
                     preferred_element_type=jnp.float32)
    p_ref[0] = p
    c_ref[0] = x1[:, :NPOINT]


def _conv1(xyz, points, W0):
    B, _, N = xyz.shape
    D = W0.shape[0]
    return pl.pallas_call(
        _conv1_body,
        grid=(B,),
        in_specs=[
            pl.BlockSpec((1, 3, N), lambda i: (i, 0, 0)),
            pl.BlockSpec((1, points.shape[1], N), lambda i: (i, 0, 0)),
            pl.BlockSpec((D, 3), lambda i: (0, 0)),
            pl.BlockSpec((D, points.shape[1]), lambda i: (0, 0)),
        ],
        out_specs=[
            pl.BlockSpec((1, D, N), lambda i: (i, 0, 0)),
            pl.BlockSpec((1, D, NPOINT), lambda i: (i, 0, 0)),
        ],
        out_shape=[
            jax.ShapeDtypeStruct((B, D, N), jnp.float32),
            jax.ShapeDtypeStruct((B, D, NPOINT), jnp.float32),
        ],
    )(xyz, points, W0[:, :3], W0[:, 3:])


def _dist_seg_body(cent_ref, xyz_ref, d_ref, seg_ref, m_sc):
    xyz = xyz_ref[0]
    cent = cent_ref[0]
    xsq = xyz[0] * xyz[0] + xyz[1] * xyz[1] + xyz[2] * xyz[2]
    csq = cent[0] * cent[0] + cent[1] * cent[1] + cent[2] * cent[2]
    xb = xyz.astype(jnp.bfloat16).astype(jnp.float32)
    cb = cent.astype(jnp.bfloat16).astype(jnp.float32)
    n = xyz.shape[1]
    for c0 in range(0, n, NCHUNK):
        c1 = c0 + NCHUNK
        d = (cb[0][:, None] * xb[0][None, c0:c1]
             + cb[1][:, None] * xb[1][None, c0:c1]
             + cb[2][:, None] * xb[2][None, c0:c1])
        d = -2.0 * d + csq[:, None] + xsq[None, c0:c1]
        d_ref[0, :, c0:c1] = d
        m = jnp.min(d.reshape(SB, NCHUNK // SEG, SEG), axis=2)
        m_sc[:, (c0 // SEG):(c1 // SEG)] = m

    nseg = n // SEG
    m = m_sc[...]
    lane = lax.broadcasted_iota(jnp.int32, (SB, nseg), 1)
    big = jnp.int32(nseg + 1)
    for k in range(NSAMPLE):
        mn = jnp.min(m, axis=1, keepdims=True)
        a = jnp.min(jnp.where(m == mn, lane, big), axis=1)
        seg_ref[0, :, k] = a
        m = jnp.where(lane == a[:, None], jnp.inf, m)


def _dist_seg(xyz):
    B, _, N = xyz.shape
    S = NPOINT
    return pl.pallas_call(
        _dist_seg_body,
        grid=(B, S // SB),
        in_specs=[
            pl.BlockSpec((1, 3, SB), lambda i, j: (i, 0, j)),
            pl.BlockSpec((1, 3, N), lambda i, j: (i, 0, 0)),
        ],
        out_specs=[
            pl.BlockSpec((1, SB, N), lambda i, j: (i, j, 0)),
            pl.BlockSpec((1, SB, NSAMPLE), lambda i, j: (i, j, 0)),
        ],
        out_shape=[
            jax.ShapeDtypeStruct((B, S, N), jnp.float32),
            jax.ShapeDtypeStruct((B, S, NSAMPLE), jnp.int32),
        ],
        scratch_shapes=[pltpu.VMEM((SB, N // SEG), jnp.float32)],
    )(xyz[..., :S], xyz)


def _topk_cand_body(cand_ref, seg_ref, idx_ref):
    v = cand_ref[0]
    seg = seg_ref[0]
    gidx = (seg[:, :, None] * SEG
            + lax.broadcasted_iota(jnp.int32, (SB, NSAMPLE, SEG), 2))
    gidx = gidx.reshape(SB, NSAMPLE * SEG)
    big = jnp.int32(1 << 30)
    for k in range(NSAMPLE):
        mn = jnp.min(v, axis=1, keepdims=True)
        a = jnp.min(jnp.where(v == mn, gidx, big), axis=1)
        idx_ref[0, :, k] = a
        v = jnp.where(gidx == a[:, None], jnp.inf, v)


def _topk_cand(cand, seg):
    B, S, _ = cand.shape
    return pl.pallas_call(
        _topk_cand_body,
        grid=(B, S // SB),
        in_specs=[
            pl.BlockSpec((1, SB, NSAMPLE * SEG), lambda i, j: (i, j, 0)),
            pl.BlockSpec((1, SB, NSAMPLE), lambda i, j: (i, j, 0)),
        ],
        out_specs=pl.BlockSpec((1, SB, NSAMPLE), lambda i, j: (i, j, 0)),
        out_shape=jax.ShapeDtypeStruct((B, S, NSAMPLE), jnp.int32),
    )(cand, seg)


def _mlp_pool_body(f_ref, c_ref, w1t_ref, o_ref):
    f = f_ref[0]
    c = c_ref[0]
    h1 = f - c[:, None, :]
    h1 = jnp.where(h1 >= 0, h1, 0.1 * h1)
    sb, k, d = h1.shape
    h2 = jnp.dot(h1.reshape(sb * k, d), w1t_ref[...],
                 preferred_element_type=jnp.float32)
    h2 = jnp.where(h2 >= 0, h2, 0.1 * h2)
    o_ref[0] = jnp.max(h2.reshape(sb, k, -1), axis=1)


def _mlp_pool(f, c, w1t, s_block=256):
    b, s, k, d = f.shape
    return pl.pallas_call(
        _mlp_pool_body,
        grid=(b, s // s_block),
        in_specs=[
            pl.BlockSpec((1, s_block, k, d), lambda i, j: (i, j, 0, 0)),
            pl.BlockSpec((1, s_block, d), lambda i, j: (i, j, 0)),
            pl.BlockSpec((d, d), lambda i, j: (0, 0)),
        ],
        out_specs=pl.BlockSpec((1, s_block, d), lambda i, j: (i, j, 0)),
        out_shape=jax.ShapeDtypeStruct((b, s, d), jnp.float32),
    )(f, c, w1t)


def kernel(xyz, points, W0, W1):
    B, C, N = xyz.shape
    S, K = NPOINT, NSAMPLE
    new_xyz = xyz[..., :S]

    P_T, X1c = _conv1(xyz, points, W0)
    P = jnp.swapaxes(P_T, 1, 2).reshape(B * N, -1)
    Cc = jnp.swapaxes(X1c, 1, 2)

    d_all, seg = _dist_seg(xyz)

    nseg = N // SEG
    row_base = jnp.arange(B * S, dtype=jnp.int32) * nseg
    sidx = (seg.reshape(B * S, K) + row_base[:, None]).reshape(-1)
    cand = _sc_gather_rows(d_all.reshape(B * S * nseg, SEG), sidx)
    cand = cand.reshape(B, S, K * SEG)

    knn_idx = _topk_cand(cand, seg)

    gidx = (knn_idx.reshape(B, S * K)
            + (jnp.arange(B, dtype=jnp.int32) * N)[:, None]).reshape(-1)
    F = _sc_gather_rows(P, gidx).reshape(B, S, K, -1)

    out = _mlp_pool(F, Cc, W1.T)
    return (new_xyz, jnp.swapaxes(out, 1, 2))


# speedup vs baseline: 16.3593x; 1.0119x over previous
import functools

import jax
import jax.numpy as jnp
from jax import lax
from jax.experimental import pallas as pl
from jax.experimental.pallas import tpu as pltpu
from jax.experimental.pallas import tpu_sc as plsc

NPOINT = 2048
NSAMPLE = 32
SEG = 32
SB = 256
NCHUNK = 2048


def _sc_gather_rows(table, idx_flat, chunk=1024):
    M, D = table.shape
    T = idx_flat.shape[0]
    info = plsc.get_sparse_core_info()
    nw = info.num_cores * info.num_subcores
    per_w = T // nw
    assert T % nw == 0 and per_w % chunk == 0

    mesh = plsc.VectorSubcoreMesh(core_axis_name="c", subcore_axis_name="s")

    @functools.partial(
        pl.kernel, mesh=mesh,
        out_type=jax.ShapeDtypeStruct((T, D), jnp.float32),
        compiler_params=pltpu.CompilerParams(use_tc_tiling_on_sc=False),
        scratch_types=[
            pltpu.VMEM((chunk,), jnp.int32),
            pltpu.VMEM((chunk, D), jnp.float32),
            pltpu.SemaphoreType.DMA,
        ],
    )
    def k(table_hbm, idx_hbm, out_hbm, idx_v, rows_v, sem):
        wid = lax.axis_index("s") * info.num_cores + lax.axis_index("c")
        base = wid * per_w

        def body(i, carry):
            off = base + i * chunk
            pltpu.sync_copy(idx_hbm.at[pl.ds(off, chunk)], idx_v)
            pltpu.async_copy(table_hbm.at[idx_v], rows_v, sem).wait()
            pltpu.sync_copy(rows_v, out_hbm.at[pl.ds(off, chunk)])
            return carry

        lax.fori_loop(0, per_w // chunk, body, 0)

    return k(table, idx_flat)


def _conv1_body(xyz_ref, pts_ref, w0x_ref, w0p_ref, p_ref, c_ref):
    xyz = xyz_ref[0]
    x1 = (w0x_ref[:, 0:1] * xyz[0:1]
          + w0x_ref[:, 1:2] * xyz[1:2]
          + w0x_ref[:, 2:3] * xyz[2:3])
    p = x1 + jnp.dot(w0p_ref[...], pts_ref[0],
                     preferred_element_type=jnp.float32)
    n = p.shape[1]
    for c0 in range(0, n, 1024):
        p_ref[0, c0:c0 + 1024] = p[:, c0:c0 + 1024].T
        if c0 < NPOINT:
            c_ref[0, c0:c0 + 1024] = x1[:, c0:c0 + 1024].T


def _conv1(xyz, points, W0):
    B, _, N = xyz.shape
    D = W0.shape[0]
    return pl.pallas_call(
        _conv1_body,
        grid=(B,),
        in_specs=[
            pl.BlockSpec((1, 3, N), lambda i: (i, 0, 0)),
            pl.BlockSpec((1, points.shape[1], N), lambda i: (i, 0, 0)),
            pl.BlockSpec((D, 3), lambda i: (0, 0)),
            pl.BlockSpec((D, points.shape[1]), lambda i: (0, 0)),
        ],
        out_specs=[
            pl.BlockSpec((1, N, D), lambda i: (i, 0, 0)),
            pl.BlockSpec((1, NPOINT, D), lambda i: (i, 0, 0)),
        ],
        out_shape=[
            jax.ShapeDtypeStruct((B, N, D), jnp.float32),
            jax.ShapeDtypeStruct((B, NPOINT, D), jnp.float32),
        ],
    )(xyz, points, W0[:, :3], W0[:, 3:])


def _dist_seg_body(cent_ref, xyz_ref, d_ref, seg_ref, m_sc):
    xyz = xyz_ref[0]
    cent = cent_ref[0]
    xsq = xyz[0] * xyz[0] + xyz[1] * xyz[1] + xyz[2] * xyz[2]
    csq = cent[0] * cent[0] + cent[1] * cent[1] + cent[2] * cent[2]
    xb = xyz.astype(jnp.bfloat16).astype(jnp.float32)
    cb = cent.astype(jnp.bfloat16).astype(jnp.float32)
    n = xyz.shape[1]
    for c0 in range(0, n, NCHUNK):
        c1 = c0 + NCHUNK
        d = (cb[0][:, None] * xb[0][None, c0:c1]
             + cb[1][:, None] * xb[1][None, c0:c1]
             + cb[2][:, None] * xb[2][None, c0:c1])
        d = -2.0 * d + csq[:, None] + xsq[None, c0:c1]
        d_ref[0, :, c0:c1] = d
        m = jnp.min(d.reshape(SB, NCHUNK // SEG, SEG), axis=2)
        m_sc[:, (c0 // SEG):(c1 // SEG)] = m

    nseg = n // SEG
    m = m_sc[...]
    lane = lax.broadcasted_iota(jnp.int32, (SB, nseg), 1)
    big = jnp.int32(nseg + 1)
    for k in range(NSAMPLE):
        mn = jnp.min(m, axis=1, keepdims=True)
        a = jnp.min(jnp.where(m == mn, lane, big), axis=1)
        seg_ref[0, :, k] = a
        m = jnp.where(lane == a[:, None], jnp.inf, m)


def _dist_seg(xyz):
    B, _, N = xyz.shape
    S = NPOINT
    return pl.pallas_call(
        _dist_seg_body,
        grid=(B, S // SB),
        in_specs=[
            pl.BlockSpec((1, 3, SB), lambda i, j: (i, 0, j)),
            pl.BlockSpec((1, 3, N), lambda i, j: (i, 0, 0)),
        ],
        out_specs=[
            pl.BlockSpec((1, SB, N), lambda i, j: (i, j, 0)),
            pl.BlockSpec((1, SB, NSAMPLE), lambda i, j: (i, j, 0)),
        ],
        out_shape=[
            jax.ShapeDtypeStruct((B, S, N), jnp.float32),
            jax.ShapeDtypeStruct((B, S, NSAMPLE), jnp.int32),
        ],
        scratch_shapes=[pltpu.VMEM((SB, N // SEG), jnp.float32)],
    )(xyz[..., :S], xyz)


def _topk_cand_body(cand_ref, seg_ref, idx_ref):
    v = cand_ref[0]
    seg = seg_ref[0]
    gidx = (seg[:, :, None] * SEG
            + lax.broadcasted_iota(jnp.int32, (SB, NSAMPLE, SEG), 2))
    gidx = gidx.reshape(SB, NSAMPLE * SEG)
    big = jnp.int32(1 << 30)
    for k in range(NSAMPLE):
        mn = jnp.min(v, axis=1, keepdims=True)
        a = jnp.min(jnp.where(v == mn, gidx, big), axis=1)
        idx_ref[0, :, k] = a
        v = jnp.where(gidx == a[:, None], jnp.inf, v)


def _topk_cand(cand, seg):
    B, S, _ = cand.shape
    return pl.pallas_call(
        _topk_cand_body,
        grid=(B, S // SB),
        in_specs=[
            pl.BlockSpec((1, SB, NSAMPLE * SEG), lambda i, j: (i, j, 0)),
            pl.BlockSpec((1, SB, NSAMPLE), lambda i, j: (i, j, 0)),
        ],
        out_specs=pl.BlockSpec((1, SB, NSAMPLE), lambda i, j: (i, j, 0)),
        out_shape=jax.ShapeDtypeStruct((B, S, NSAMPLE), jnp.int32),
    )(cand, seg)


def _mlp_pool_body(f_ref, c_ref, w1t_ref, o_ref):
    f = f_ref[0]
    c = c_ref[0]
    h1 = f - c[:, None, :]
    h1 = jnp.where(h1 >= 0, h1, 0.1 * h1)
    sb, k, d = h1.shape
    h2 = jnp.dot(h1.reshape(sb * k, d), w1t_ref[...],
                 preferred_element_type=jnp.float32)
    h2 = jnp.where(h2 >= 0, h2, 0.1 * h2)
    o_ref[0] = jnp.max(h2.reshape(sb, k, -1), axis=1)


def _mlp_pool(f, c, w1t, s_block=256):
    b, s, k, d = f.shape
    return pl.pallas_call(
        _mlp_pool_body,
        grid=(b, s // s_block),
        in_specs=[
            pl.BlockSpec((1, s_block, k, d), lambda i, j: (i, j, 0, 0)),
            pl.BlockSpec((1, s_block, d), lambda i, j: (i, j, 0)),
            pl.BlockSpec((d, d), lambda i, j: (0, 0)),
        ],
        out_specs=pl.BlockSpec((1, s_block, d), lambda i, j: (i, j, 0)),
        out_shape=jax.ShapeDtypeStruct((b, s, d), jnp.float32),
    )(f, c, w1t)


def kernel(xyz, points, W0, W1):
    B, C, N = xyz.shape
    S, K = NPOINT, NSAMPLE
    new_xyz = xyz[..., :S]

    P, Cc = _conv1(xyz, points, W0)
    P = P.reshape(B * N, -1)

    d_all, seg = _dist_seg(xyz)

    nseg = N // SEG
    row_base = jnp.arange(B * S, dtype=jnp.int32) * nseg
    sidx = (seg.reshape(B * S, K) + row_base[:, None]).reshape(-1)
    cand = _sc_gather_rows(d_all.reshape(B * S * nseg, SEG), sidx)
    cand = cand.reshape(B, S, K * SEG)

    knn_idx = _topk_cand(cand, seg)

    gidx = (knn_idx.reshape(B, S * K)
            + (jnp.arange(B, dtype=jnp.int32) * N)[:, None]).reshape(-1)
    F = _sc_gather_rows(P, gidx).reshape(B, S, K, -1)

    out = _mlp_pool(F, Cc, W1.T)
    return (new_xyz, jnp.swapaxes(out, 1, 2))


# speedup vs baseline: 24.7179x; 1.5109x over previous
import functools

import jax
import jax.numpy as jnp
from jax import lax
from jax.experimental import pallas as pl
from jax.experimental.pallas import tpu as pltpu
from jax.experimental.pallas import tpu_sc as plsc

NPOINT = 2048
NSAMPLE = 32
SEG = 32
SB = 256
NCHUNK = 2048


def _sc_gather_rows(table, idx_flat, chunk=1024):
    M, D = table.shape
    T = idx_flat.shape[0]
    info = plsc.get_sparse_core_info()
    nw = info.num_cores * info.num_subcores
    per_w = T // nw
    assert T % nw == 0 and per_w % chunk == 0

    mesh = plsc.VectorSubcoreMesh(core_axis_name="c", subcore_axis_name="s")

    @functools.partial(
        pl.kernel, mesh=mesh,
        out_type=jax.ShapeDtypeStruct((T, D), jnp.float32),
        compiler_params=pltpu.CompilerParams(use_tc_tiling_on_sc=False),
        scratch_types=[
            pltpu.VMEM((chunk,), jnp.int32),
            pltpu.VMEM((chunk, D), jnp.float32),
            pltpu.SemaphoreType.DMA,
        ],
    )
    def k(table_hbm, idx_hbm, out_hbm, idx_v, rows_v, sem):
        wid = lax.axis_index("s") * info.num_cores + lax.axis_index("c")
        base = wid * per_w

        def body(i, carry):
            off = base + i * chunk
            pltpu.sync_copy(idx_hbm.at[pl.ds(off, chunk)], idx_v)
            pltpu.async_copy(table_hbm.at[idx_v], rows_v, sem).wait()
            pltpu.sync_copy(rows_v, out_hbm.at[pl.ds(off, chunk)])
            return carry

        lax.fori_loop(0, per_w // chunk, body, 0)

    return k(table, idx_flat)


def _conv1_body(xyz_ref, pts_ref, w0x_ref, w0p_ref, p_ref, c_ref):
    xyz = xyz_ref[0]
    x1 = (w0x_ref[:, 0:1] * xyz[0:1]
          + w0x_ref[:, 1:2] * xyz[1:2]
          + w0x_ref[:, 2:3] * xyz[2:3])
    p = x1 + jnp.dot(w0p_ref[...], pts_ref[0],
                     preferred_element_type=jnp.float32)
    n = p.shape[1]
    for c0 in range(0, n, 1024):
        p_ref[0, c0:c0 + 1024] = p[:, c0:c0 + 1024].T
        if c0 < NPOINT:
            c_ref[0, c0:c0 + 1024] = x1[:, c0:c0 + 1024].T


def _conv1(xyz, points, W0):
    B, _, N = xyz.shape
    D = W0.shape[0]
    return pl.pallas_call(
        _conv1_body,
        grid=(B,),
        in_specs=[
            pl.BlockSpec((1, 3, N), lambda i: (i, 0, 0)),
            pl.BlockSpec((1, points.shape[1], N), lambda i: (i, 0, 0)),
            pl.BlockSpec((D, 3), lambda i: (0, 0)),
            pl.BlockSpec((D, points.shape[1]), lambda i: (0, 0)),
        ],
        out_specs=[
            pl.BlockSpec((1, N, D), lambda i: (i, 0, 0)),
            pl.BlockSpec((1, NPOINT, D), lambda i: (i, 0, 0)),
        ],
        out_shape=[
            jax.ShapeDtypeStruct((B, N, D), jnp.float32),
            jax.ShapeDtypeStruct((B, NPOINT, D), jnp.float32),
        ],
    )(xyz, points, W0[:, :3], W0[:, 3:])


def _dist_seg_body(cent_ref, xyz_ref, d_ref, seg_ref, m_sc):
    xyz = xyz_ref[0]
    cent = cent_ref[0]
    xsq = xyz[0] * xyz[0] + xyz[1] * xyz[1] + xyz[2] * xyz[2]
    csq = cent[0] * cent[0] + cent[1] * cent[1] + cent[2] * cent[2]
    xb = xyz.astype(jnp.bfloat16).astype(jnp.float32)
    cb = cent.astype(jnp.bfloat16).astype(jnp.float32)
    n = xyz.shape[1]
    for c0 in range(0, n, NCHUNK):
        c1 = c0 + NCHUNK
        d = (cb[0][:, None] * xb[0][None, c0:c1]
             + cb[1][:, None] * xb[1][None, c0:c1]
             + cb[2][:, None] * xb[2][None, c0:c1])
        d = -2.0 * d + csq[:, None] + xsq[None, c0:c1]
        d_ref[0, :, c0:c1] = d
        t = jnp.transpose(d)
        mt = jnp.min(t.reshape(NCHUNK // SEG, SEG, SB), axis=1)
        m_sc[(c0 // SEG):(c1 // SEG), :] = mt

    nseg = n // SEG
    m = m_sc[...]
    sub = lax.broadcasted_iota(jnp.int32, (nseg, SB), 0)
    big = jnp.int32(nseg + 1)
    for k in range(NSAMPLE):
        mn = jnp.min(m, axis=0)
        a = jnp.min(jnp.where(m == mn[None, :], sub, big), axis=0)
        seg_ref[0, k] = a
        m = jnp.where(sub == a[None, :], jnp.inf, m)


def _dist_seg(xyz):
    B, _, N = xyz.shape
    S = NPOINT
    return pl.pallas_call(
        _dist_seg_body,
        grid=(B, S // SB),
        in_specs=[
            pl.BlockSpec((1, 3, SB), lambda i, j: (i, 0, j)),
            pl.BlockSpec((1, 3, N), lambda i, j: (i, 0, 0)),
        ],
        out_specs=[
            pl.BlockSpec((1, SB, N), lambda i, j: (i, j, 0)),
            pl.BlockSpec((1, NSAMPLE, SB), lambda i, j: (i, 0, j)),
        ],
        out_shape=[
            jax.ShapeDtypeStruct((B, S, N), jnp.float32),
            jax.ShapeDtypeStruct((B, NSAMPLE, S), jnp.int32),
        ],
        scratch_shapes=[pltpu.VMEM((N // SEG, SB), jnp.float32)],
    )(xyz[..., :S], xyz)


def _topk_cand_body(cand_ref, seg_ref, idx_ref):
    v = cand_ref[0]
    seg = seg_ref[0]
    gidx = (seg[:, :, None] * SEG
            + lax.broadcasted_iota(jnp.int32, (SB, NSAMPLE, SEG), 2))
    gidx = gidx.reshape(SB, NSAMPLE * SEG)
    big = jnp.int32(1 << 30)
    for k in range(NSAMPLE):
        mn = jnp.min(v, axis=1, keepdims=True)
        a = jnp.min(jnp.where(v == mn, gidx, big), axis=1)
        idx_ref[0, :, k] = a
        v = jnp.where(gidx == a[:, None], jnp.inf, v)


def _topk_cand(cand, seg):
    B, S, _ = cand.shape
    return pl.pallas_call(
        _topk_cand_body,
        grid=(B, S // SB),
        in_specs=[
            pl.BlockSpec((1, SB, NSAMPLE * SEG), lambda i, j: (i, j, 0)),
            pl.BlockSpec((1, SB, NSAMPLE), lambda i, j: (i, j, 0)),
        ],
        out_specs=pl.BlockSpec((1, SB, NSAMPLE), lambda i, j: (i, j, 0)),
        out_shape=jax.ShapeDtypeStruct((B, S, NSAMPLE), jnp.int32),
    )(cand, seg)


def _mlp_pool_body(f_ref, c_ref, w1t_ref, o_ref):
    f = f_ref[0]
    c = c_ref[0]
    h1 = f - c[:, None, :]
    h1 = jnp.where(h1 >= 0, h1, 0.1 * h1)
    sb, k, d = h1.shape
    h2 = jnp.dot(h1.reshape(sb * k, d), w1t_ref[...],
                 preferred_element_type=jnp.float32)
    h2 = jnp.where(h2 >= 0, h2, 0.1 * h2)
    o_ref[0] = jnp.max(h2.reshape(sb, k, -1), axis=1)


def _mlp_pool(f, c, w1t, s_block=256):
    b, s, k, d = f.shape
    return pl.pallas_call(
        _mlp_pool_body,
        grid=(b, s // s_block),
        in_specs=[
            pl.BlockSpec((1, s_block, k, d), lambda i, j: (i, j, 0, 0)),
            pl.BlockSpec((1, s_block, d), lambda i, j: (i, j, 0)),
            pl.BlockSpec((d, d), lambda i, j: (0, 0)),
        ],
        out_specs=pl.BlockSpec((1, s_block, d), lambda i, j: (i, j, 0)),
        out_shape=jax.ShapeDtypeStruct((b, s, d), jnp.float32),
    )(f, c, w1t)


def kernel(xyz, points, W0, W1):
    B, C, N = xyz.shape
    S, K = NPOINT, NSAMPLE
    new_xyz = xyz[..., :S]

    P, Cc = _conv1(xyz, points, W0)
    P = P.reshape(B * N, -1)

    d_all, seg_t = _dist_seg(xyz)
    seg = jnp.swapaxes(seg_t, 1, 2)

    nseg = N // SEG
    row_base = jnp.arange(B * S, dtype=jnp.int32) * nseg
    sidx = (seg.reshape(B * S, K) + row_base[:, None]).reshape(-1)
    cand = _sc_gather_rows(d_all.reshape(B * S * nseg, SEG), sidx)
    cand = cand.reshape(B, S, K * SEG)

    knn_idx = _topk_cand(cand, seg)

    gidx = (knn_idx.reshape(B, S * K)
            + (jnp.arange(B, dtype=jnp.int32) * N)[:, None]).reshape(-1)
    F = _sc_gather_rows(P, gidx).reshape(B, S, K, -1)

    out = _mlp_pool(F, Cc, W1.T)
    return (new_xyz, jnp.swapaxes(out, 1, 2))


# speedup vs baseline: 25.9822x; 1.0512x over previous
import functools

import jax
import jax.numpy as jnp
from jax import lax
from jax.experimental import pallas as pl
from jax.experimental.pallas import tpu as pltpu
from jax.experimental.pallas import tpu_sc as plsc

NPOINT = 2048
NSAMPLE = 32
SEG = 32
SB = 256
NCHUNK = 2048


def _sc_gather_rows(table, idx_flat, chunk=1024):
    M, D = table.shape
    T = idx_flat.shape[0]
    info = plsc.get_sparse_core_info()
    nw = info.num_cores * info.num_subcores
    per_w = T // nw
    assert T % nw == 0 and per_w % chunk == 0

    mesh = plsc.VectorSubcoreMesh(core_axis_name="c", subcore_axis_name="s")

    @functools.partial(
        pl.kernel, mesh=mesh,
        out_type=jax.ShapeDtypeStruct((T, D), jnp.float32),
        compiler_params=pltpu.CompilerParams(use_tc_tiling_on_sc=False),
        scratch_types=[
            pltpu.VMEM((chunk,), jnp.int32),
            pltpu.VMEM((chunk, D), jnp.float32),
            pltpu.SemaphoreType.DMA,
        ],
    )
    def k(table_hbm, idx_hbm, out_hbm, idx_v, rows_v, sem):
        wid = lax.axis_index("s") * info.num_cores + lax.axis_index("c")
        base = wid * per_w

        def body(i, carry):
            off = base + i * chunk
            pltpu.sync_copy(idx_hbm.at[pl.ds(off, chunk)], idx_v)
            pltpu.async_copy(table_hbm.at[idx_v], rows_v, sem).wait()
            pltpu.sync_copy(rows_v, out_hbm.at[pl.ds(off, chunk)])
            return carry

        lax.fori_loop(0, per_w // chunk, body, 0)

    return k(table, idx_flat)


def _conv1_body(xyz_ref, pts_ref, w0x_ref, w0p_ref, p_ref, c_ref):
    xyz = xyz_ref[0]
    x1 = (w0x_ref[:, 0:1] * xyz[0:1]
          + w0x_ref[:, 1:2] * xyz[1:2]
          + w0x_ref[:, 2:3] * xyz[2:3])
    p = x1 + jnp.dot(w0p_ref[...], pts_ref[0],
                     preferred_element_type=jnp.float32)
    n = p.shape[1]
    for c0 in range(0, n, 1024):
        p_ref[0, c0:c0 + 1024] = p[:, c0:c0 + 1024].T
        if c0 < NPOINT:
            c_ref[0, c0:c0 + 1024] = x1[:, c0:c0 + 1024].T


def _conv1(xyz, points, W0):
    B, _, N = xyz.shape
    D = W0.shape[0]
    return pl.pallas_call(
        _conv1_body,
        grid=(B,),
        in_specs=[
            pl.BlockSpec((1, 3, N), lambda i: (i, 0, 0)),
            pl.BlockSpec((1, points.shape[1], N), lambda i: (i, 0, 0)),
            pl.BlockSpec((D, 3), lambda i: (0, 0)),
            pl.BlockSpec((D, points.shape[1]), lambda i: (0, 0)),
        ],
        out_specs=[
            pl.BlockSpec((1, N, D), lambda i: (i, 0, 0)),
            pl.BlockSpec((1, NPOINT, D), lambda i: (i, 0, 0)),
        ],
        out_shape=[
            jax.ShapeDtypeStruct((B, N, D), jnp.float32),
            jax.ShapeDtypeStruct((B, NPOINT, D), jnp.float32),
        ],
    )(xyz, points, W0[:, :3], W0[:, 3:])


def _dist_seg_body(cent_ref, xyz_ref, d_ref, seg_ref, m_sc):
    xyz = xyz_ref[0]
    cent = cent_ref[0]
    xsq = xyz[0] * xyz[0] + xyz[1] * xyz[1] + xyz[2] * xyz[2]
    csq = cent[0] * cent[0] + cent[1] * cent[1] + cent[2] * cent[2]
    xb = xyz.astype(jnp.bfloat16).astype(jnp.float32)
    cb = cent.astype(jnp.bfloat16).astype(jnp.float32)
    n = xyz.shape[1]
    for c0 in range(0, n, NCHUNK):
        c1 = c0 + NCHUNK
        d = (cb[0][:, None] * xb[0][None, c0:c1]
             + cb[1][:, None] * xb[1][None, c0:c1]
             + cb[2][:, None] * xb[2][None, c0:c1])
        d = -2.0 * d + csq[:, None] + xsq[None, c0:c1]
        d_ref[0, :, c0:c1] = d
        t = jnp.transpose(d)
        mt = jnp.min(t.reshape(NCHUNK // SEG, SEG, SB), axis=1)
        m_sc[(c0 // SEG):(c1 // SEG), :] = mt

    nseg = n // SEG
    m = m_sc[...]
    sub = lax.broadcasted_iota(jnp.int32, (nseg, SB), 0)
    big = jnp.int32(nseg + 1)
    for k in range(NSAMPLE):
        mn = jnp.min(m, axis=0)
        a = jnp.min(jnp.where(m == mn[None, :], sub, big), axis=0)
        seg_ref[0, k] = a
        m = jnp.where(sub == a[None, :], jnp.inf, m)


def _dist_seg(xyz):
    B, _, N = xyz.shape
    S = NPOINT
    return pl.pallas_call(
        _dist_seg_body,
        grid=(B, S // SB),
        in_specs=[
            pl.BlockSpec((1, 3, SB), lambda i, j: (i, 0, j)),
            pl.BlockSpec((1, 3, N), lambda i, j: (i, 0, 0)),
        ],
        out_specs=[
            pl.BlockSpec((1, SB, N), lambda i, j: (i, j, 0)),
            pl.BlockSpec((1, NSAMPLE, SB), lambda i, j: (i, 0, j)),
        ],
        out_shape=[
            jax.ShapeDtypeStruct((B, S, N), jnp.float32),
            jax.ShapeDtypeStruct((B, NSAMPLE, S), jnp.int32),
        ],
        scratch_shapes=[pltpu.VMEM((N // SEG, SB), jnp.float32)],
    )(xyz[..., :S], xyz)


def _topk_cand_body(cand_ref, seg_ref, idx_ref):
    v = cand_ref[0]
    seg = seg_ref[0]
    gidx = (seg[:, :, None] * SEG
            + lax.broadcasted_iota(jnp.int32, (SB, NSAMPLE, SEG), 2))
    gidx = gidx.reshape(SB, NSAMPLE * SEG)
    big = jnp.int32(1 << 30)
    for k in range(NSAMPLE):
        mn = jnp.min(v, axis=1, keepdims=True)
        a = jnp.min(jnp.where(v == mn, gidx, big), axis=1)
        idx_ref[0, :, k] = a
        v = jnp.where(gidx == a[:, None], jnp.inf, v)


def _topk_cand(cand, seg):
    B, S, _ = cand.shape
    return pl.pallas_call(
        _topk_cand_body,
        grid=(B, S // SB),
        in_specs=[
            pl.BlockSpec((1, SB, NSAMPLE * SEG), lambda i, j: (i, j, 0)),
            pl.BlockSpec((1, SB, NSAMPLE), lambda i, j: (i, j, 0)),
        ],
        out_specs=pl.BlockSpec((1, SB, NSAMPLE), lambda i, j: (i, j, 0)),
        out_shape=jax.ShapeDtypeStruct((B, S, NSAMPLE), jnp.int32),
    )(cand, seg)


def _mlp_pool_body(f_ref, c_ref, w1t_ref, o_ref):
    f = f_ref[0]
    c = c_ref[0]
    h1 = f - c[:, None, :]
    h1 = jnp.where(h1 >= 0, h1, 0.1 * h1)
    sb, k, d = h1.shape
    h2 = jnp.dot(h1.reshape(sb * k, d), w1t_ref[...],
                 preferred_element_type=jnp.float32)
    h2 = jnp.where(h2 >= 0, h2, 0.1 * h2)
    o_ref[0] = jnp.max(h2.reshape(sb, k, -1), axis=1)


def _mlp_pool(f, c, w1t, s_block=256):
    b, s, k, d = f.shape
    return pl.pallas_call(
        _mlp_pool_body,
        grid=(b, s // s_block),
        in_specs=[
            pl.BlockSpec((1, s_block, k, d), lambda i, j: (i, j, 0, 0)),
            pl.BlockSpec((1, s_block, d), lambda i, j: (i, j, 0)),
            pl.BlockSpec((d, d), lambda i, j: (0, 0)),
        ],
        out_specs=pl.BlockSpec((1, s_block, d), lambda i, j: (i, j, 0)),
        out_shape=jax.ShapeDtypeStruct((b, s, d), jnp.float32),
    )(f, c, w1t)


def kernel(xyz, points, W0, W1):
    B, C, N = xyz.shape
    S, K = NPOINT, NSAMPLE
    new_xyz = xyz[..., :S]

    P, Cc = _conv1(xyz, points, W0)
    P = P.reshape(B * N, -1)

    nseg = N // SEG
    row_base = jnp.arange(S, dtype=jnp.int32) * nseg
    w1t = W1.T
    outs = []
    for b in range(B):
        xyz_b = lax.slice_in_dim(xyz, b, b + 1, axis=0)
        d_b, seg_t = _dist_seg(xyz_b)
        seg = jnp.swapaxes(seg_t, 1, 2)

        sidx = (seg.reshape(S, K) + row_base[:, None]).reshape(-1)
        cand = _sc_gather_rows(d_b.reshape(S * nseg, SEG), sidx)
        cand = cand.reshape(1, S, K * SEG)

        knn_idx = _topk_cand(cand, seg)

        F = _sc_gather_rows(P[b * N:(b + 1) * N],
                            knn_idx.reshape(-1)).reshape(1, S, K, -1)

        outs.append(_mlp_pool(F, lax.slice_in_dim(Cc, b, b + 1, axis=0), w1t))

    out = jnp.concatenate(outs, axis=0)
    return (new_xyz, jnp.swapaxes(out, 1, 2))
